# Initial kernel scaffold; baseline (speedup 1.0000x reference)
import jax
import jax.numpy as jnp
from jax.experimental import pallas as pl


def kernel(feat_map, keys):
    raise NotImplementedError("write your pallas kernel here")


# speedup vs baseline: 3.2671x; 3.2671x over previous
from functools import partial

import jax
import jax.numpy as jnp
from jax.experimental import pallas as pl
from jax.experimental.pallas import tpu as pltpu

_TOPK = 5
_KT = 2048


def _knn_kernel(q_ref, k_ref, mean_ref, best_ref, q2_ref, *, n_tiles):
    t = pl.program_id(0)
    q = q_ref[...]

    @pl.when(t == 0)
    def _init():
        best_ref[...] = jnp.full(best_ref.shape, jnp.inf, jnp.float32)
        q2_ref[...] = jnp.sum(q * q, axis=1, keepdims=True)

    k = k_ref[...]
    sim = jax.lax.dot_general(
        q, k, (((1,), (1,)), ((), ())), preferred_element_type=jnp.float32
    )
    ksq = k * k
    ones = jnp.ones((8, k.shape[1]), jnp.float32)
    k2all = jax.lax.dot_general(
        ones, ksq, (((1,), (1,)), ((), ())), preferred_element_type=jnp.float32
    )
    k2 = k2all[0:1, :]
    d = q2_ref[...] + k2 - 2.0 * sim

    best = best_ref[...]
    inf = jnp.float32(jnp.inf)
    vals = []
    cnts = []
    for _ in range(_TOPK):
        m = jnp.minimum(
            jnp.min(d, axis=1, keepdims=True),
            jnp.min(best, axis=1, keepdims=True),
        )
        eq_d = d == m
        eq_b = best == m
        cnt = jnp.sum(eq_d, axis=1, keepdims=True, dtype=jnp.float32) + jnp.sum(
            eq_b, axis=1, keepdims=True, dtype=jnp.float32
        )
        d = jnp.where(eq_d, inf, d)
        best = jnp.where(eq_b, inf, best)
        vals.append(m)
        cnts.append(cnt)

    cum = []
    c = jnp.zeros_like(cnts[0])
    for i in range(_TOPK):
        c = c + cnts[i]
        cum.append(c)
    new_cols = []
    for j in range(_TOPK):
        jj = jnp.float32(j)
        v = vals[_TOPK - 1]
        for i in range(_TOPK - 2, -1, -1):
            v = jnp.where(cum[i] > jj, vals[i], v)
        new_cols.append(v)
    for j in range(_TOPK):
        best_ref[:, j : j + 1] = new_cols[j]

    s = new_cols[0]
    for j in range(1, _TOPK):
        s = s + new_cols[j]
    mean_ref[...] = s * jnp.float32(1.0 / _TOPK)


def _norm_kernel(x_ref, o_ref):
    x = x_ref[...]
    vmin = jnp.min(x, axis=1, keepdims=True)
    vmax = jnp.max(x, axis=1, keepdims=True)
    o_ref[...] = (x - vmin) / (vmax - vmin + jnp.float32(1e-6))


@jax.jit
def kernel(feat_map, keys):
    B, C, H, W = feat_map.shape
    q = jnp.transpose(feat_map, (0, 2, 3, 1)).reshape(-1, C)
    N = q.shape[0]
    M = keys.shape[0]

    n_tiles = (M + _KT - 1) // _KT
    m_pad = n_tiles * _KT
    if m_pad != M:
        keys = jnp.concatenate(
            [keys, jnp.full((m_pad - M, C), 1e4, dtype=keys.dtype)], axis=0
        )

    mean = pl.pallas_call(
        partial(_knn_kernel, n_tiles=n_tiles),
        grid=(n_tiles,),
        in_specs=[
            pl.BlockSpec((N, C), lambda t: (0, 0)),
            pl.BlockSpec((_KT, C), lambda t: (t, 0)),
        ],
        out_specs=pl.BlockSpec((N, 1), lambda t: (0, 0)),
        out_shape=jax.ShapeDtypeStruct((N, 1), jnp.float32),
        scratch_shapes=[
            pltpu.VMEM((N, 128), jnp.float32),
            pltpu.VMEM((N, 1), jnp.float32),
        ],
        compiler_params=pltpu.CompilerParams(dimension_semantics=("arbitrary",)),
    )(q, keys)

    hw = H * W
    tm = mean.reshape(B, hw)
    lanes = ((hw + 127) // 128) * 128
    tmp = jnp.concatenate(
        [tm, jnp.broadcast_to(tm[:, :1], (B, lanes - hw))], axis=1
    )
    amap = pl.pallas_call(
        _norm_kernel,
        out_shape=jax.ShapeDtypeStruct((B, lanes), jnp.float32),
    )(tmp)
    return amap[:, :hw].reshape(B, H, W)


# speedup vs baseline: 4.4653x; 1.3667x over previous
from functools import partial

import jax
import jax.numpy as jnp
from jax.experimental import pallas as pl
from jax.experimental.pallas import tpu as pltpu

_TOPK = 5
_KT = 2048


def _extract5(arrs):
    inf = jnp.float32(jnp.inf)
    vals, cnts = [], []
    for _ in range(_TOPK):
        m = None
        for a in arrs:
            am = jnp.min(a, axis=1, keepdims=True)
            m = am if m is None else jnp.minimum(m, am)
        cnt = None
        eqs = []
        for a in arrs:
            eq = a == m
            eqs.append(eq)
            c = jnp.sum(eq, axis=1, keepdims=True, dtype=jnp.float32)
            cnt = c if cnt is None else cnt + c
        arrs = [jnp.where(eq, inf, a) for a, eq in zip(arrs, eqs)]
        vals.append(m)
        cnts.append(cnt)
    return vals, cnts


def _rebuild(vals, cnts):
    cum = []
    c = jnp.zeros_like(cnts[0])
    for i in range(_TOPK):
        c = c + cnts[i]
        cum.append(c)
    cols = []
    for j in range(_TOPK):
        jj = jnp.float32(j)
        v = vals[_TOPK - 1]
        for i in range(_TOPK - 2, -1, -1):
            v = jnp.where(cum[i] > jj, vals[i], v)
        cols.append(v)
    return cols


def _knn_kernel(q_ref, k_ref, mean_ref, best_ref, q2_ref, *, n_tiles):
    t = pl.program_id(0)
    q = q_ref[...]

    @pl.when(t == 0)
    def _init():
        best_ref[...] = jnp.full(best_ref.shape, jnp.inf, jnp.float32)
        q2_ref[...] = jnp.sum(q * q, axis=1, keepdims=True)

    k = k_ref[...]
    sim = jax.lax.dot_general(
        q, k, (((1,), (1,)), ((), ())), preferred_element_type=jnp.float32
    )
    ksq = k * k
    ones = jnp.ones((8, k.shape[1]), jnp.float32)
    k2all = jax.lax.dot_general(
        ones, ksq, (((1,), (1,)), ((), ())), preferred_element_type=jnp.float32
    )
    k2 = k2all[0:1, :]
    d = q2_ref[...] + k2 - 2.0 * sim

    best = best_ref[...]
    inf = jnp.float32(jnp.inf)

    g_cnt = d.shape[1] // 128
    a1 = d[:, 0:128]
    a2 = jnp.full(a1.shape, jnp.inf, jnp.float32)
    a3 = a2
    for g in range(1, g_cnt):
        s = d[:, g * 128 : (g + 1) * 128]
        lo = jnp.minimum(a1, s)
        hi = jnp.maximum(a1, s)
        a1 = lo
        lo2 = jnp.minimum(a2, hi)
        hi2 = jnp.maximum(a2, hi)
        a2 = lo2
        a3 = jnp.minimum(a3, hi2)

    cand = jnp.concatenate([a1, a2, a3, best], axis=1)
    vals, cnts = _extract5([cand])
    cols = _rebuild(vals, cnts)
    v5 = cols[_TOPK - 1]

    e = None
    for j in range(_TOPK):
        ej = (cols[j] < v5).astype(jnp.float32)
        e = ej if e is None else e + ej
    cnt_full = jnp.sum(
        (d < v5).astype(jnp.float32), axis=1, keepdims=True
    ) + jnp.sum((best < v5).astype(jnp.float32), axis=1, keepdims=True)
    anybad = jnp.max(jnp.abs(cnt_full - e)) > 0.0

    def _write(cols_out):
        for j in range(_TOPK):
            best_ref[:, j : j + 1] = cols_out[j]
        s = cols_out[0]
        for j in range(1, _TOPK):
            s = s + cols_out[j]
        mean_ref[...] = s * jnp.float32(1.0 / _TOPK)

    @pl.when(jnp.logical_not(anybad))
    def _fast():
        _write(cols)

    @pl.when(anybad)
    def _slow():
        vals_s, cnts_s = _extract5([d, best])
        _write(_rebuild(vals_s, cnts_s))


def _norm_kernel(x_ref, o_ref):
    x = x_ref[...]
    vmin = jnp.min(x, axis=1, keepdims=True)
    vmax = jnp.max(x, axis=1, keepdims=True)
    o_ref[...] = (x - vmin) / (vmax - vmin + jnp.float32(1e-6))


@jax.jit
def kernel(feat_map, keys):
    B, C, H, W = feat_map.shape
    q = jnp.transpose(feat_map, (0, 2, 3, 1)).reshape(-1, C)
    N = q.shape[0]
    M = keys.shape[0]

    n_tiles = (M + _KT - 1) // _KT
    m_pad = n_tiles * _KT
    if m_pad != M:
        keys = jnp.concatenate(
            [keys, jnp.full((m_pad - M, C), 1e4, dtype=keys.dtype)], axis=0
        )

    mean = pl.pallas_call(
        partial(_knn_kernel, n_tiles=n_tiles),
        grid=(n_tiles,),
        in_specs=[
            pl.BlockSpec((N, C), lambda t: (0, 0)),
            pl.BlockSpec((_KT, C), lambda t: (t, 0)),
        ],
        out_specs=pl.BlockSpec((N, 1), lambda t: (0, 0)),
        out_shape=jax.ShapeDtypeStruct((N, 1), jnp.float32),
        scratch_shapes=[
            pltpu.VMEM((N, 128), jnp.float32),
            pltpu.VMEM((N, 1), jnp.float32),
        ],
        compiler_params=pltpu.CompilerParams(dimension_semantics=("arbitrary",)),
    )(q, keys)

    hw = H * W
    tm = mean.reshape(B, hw)
    lanes = ((hw + 127) // 128) * 128
    tmp = jnp.concatenate(
        [tm, jnp.broadcast_to(tm[:, :1], (B, lanes - hw))], axis=1
    )
    amap = pl.pallas_call(
        _norm_kernel,
        out_shape=jax.ShapeDtypeStruct((B, lanes), jnp.float32),
    )(tmp)
    return amap[:, :hw].reshape(B, H, W)
